# Initial kernel scaffold; baseline (speedup 1.0000x reference)
import jax
import jax.numpy as jnp
from jax.experimental import pallas as pl


def kernel(x, edge_index, W1, a_src1, a_dst1, b1, bn_g, bn_b, bn_rm, bn_rv, W2, a_src2, a_dst2, b2, Wf, bf):
    raise NotImplementedError("write your pallas kernel here")


# speedup vs baseline: 44.7344x; 44.7344x over previous
import jax
import jax.numpy as jnp
from jax import lax
from jax.experimental import pallas as pl
from jax.experimental.pallas import tpu as pltpu
from jax.experimental.pallas import tpu_sc as plsc

N = 10000
HID = 64
HEADS = 2
OUT_CH = 128

NPD = 10240
STRIPE = NPD // 16
ZC = 64
CH = 64
ET = N + 320000
EPT = 20736
NCHUNK = EPT // CH
EPAD = 16 * EPT


def _k1_body(x_ref, w_ref, a_ref, h_ref, asad_ref, m_ref):
    h = jnp.dot(x_ref[...], w_ref[...], preferred_element_type=jnp.float32)
    h_ref[0, :, :] = h[:, 0:HID]
    h_ref[1, :, :] = h[:, HID:2 * HID]
    asad = jnp.dot(h, a_ref[...], preferred_element_type=jnp.float32)
    asad_ref[...] = asad
    mx = jnp.max(asad, axis=0, keepdims=True)
    ms = mx[:, 0:2] + mx[:, 2:4]
    ms = jnp.where(ms > 0, ms, 0.2 * ms)
    m_ref[...] = jnp.concatenate(
        [ms, jnp.zeros((1, 14), jnp.float32)], axis=1)


def _proj(x, w, a):
    n = x.shape[0]
    return pl.pallas_call(
        _k1_body,
        out_shape=[
            jax.ShapeDtypeStruct((HEADS, n, HID), jnp.float32),
            jax.ShapeDtypeStruct((n, 8), jnp.float32),
            jax.ShapeDtypeStruct((1, 16), jnp.float32),
        ],
    )(x, w, a)


_sc_mesh = plsc.VectorSubcoreMesh(core_axis_name="c", subcore_axis_name="s")


def _sc_edge_body(h_hbm, tbl_hbm, mv_hbm, src_hbm, dst_hbm,
                  num_out, den_out,
                  tbl_v, den_v, sslab, dslab, rows_v, ibuf, pbuf, mv,
                  num_sh, sem):
    c = lax.axis_index("c")
    s = lax.axis_index("s")

    def _zb(j, carry):
        for cc in range(HID // 16):
            rows_v[j, pl.ds(cc * 16, 16)] = jnp.zeros((16,), jnp.float32)
        return carry
    lax.fori_loop(0, ZC, _zb, 0)

    def _zn(k, carry):
        pltpu.sync_copy(rows_v, num_sh.at[pl.ds(s * STRIPE + k * ZC, ZC)])
        return carry
    lax.fori_loop(0, STRIPE // ZC, _zn, 0)

    def _zd(i, carry):
        den_v[pl.ds(i * 16, 16)] = jnp.zeros((16,), jnp.float32)
        return carry
    lax.fori_loop(0, NPD // 16, _zd, 0)

    pltpu.sync_copy(tbl_hbm.at[c], tbl_v)
    pltpu.sync_copy(mv_hbm, mv)
    pltpu.sync_copy(src_hbm.at[s], sslab)
    pltpu.sync_copy(dst_hbm.at[s], dslab)
    mvv = mv[pl.ds(0, 16)]
    m = jnp.where(c == 0, mvv[0], mvv[1])
    hoff = c * NPD
    plsc.subcore_barrier()

    def _chunk(ci, carry):
        for g in range(CH // 16):
            s16 = sslab[ci, pl.ds(g * 16, 16)]
            d16 = dslab[ci, pl.ds(g * 16, 16)]
            ibuf[pl.ds(g * 16, 16)] = s16 + hoff
            a_s = plsc.load_gather(tbl_v, [s16])
            a_d = plsc.load_gather(tbl_v, [d16 + NPD])
            e = a_s + a_d
            e = jnp.where(e > 0, e, 0.2 * e) - m
            p = jnp.exp(e)
            plsc.addupdate_scatter(den_v, [d16], p)
            pbuf[0, pl.ds(g * 16, 16)] = p
        pltpu.async_copy(h_hbm.at[ibuf], rows_v, sem).wait()

        def _scale(j, carry2):
            ps = pbuf[0, pl.ds(j, 16)][0]
            for cc in range(HID // 16):
                rows_v[j, pl.ds(cc * 16, 16)] = (
                    rows_v[j, pl.ds(cc * 16, 16)] * ps)
            return carry2
        lax.fori_loop(0, CH, _scale, 0)
        pltpu.sync_copy(rows_v, num_sh.at[dslab.at[ci]], add=True)
        return carry
    lax.fori_loop(0, NCHUNK, _chunk, 0)
    plsc.subcore_barrier()

    def _ex(k, carry):
        r0 = s * STRIPE + k * ZC
        pltpu.sync_copy(num_sh.at[pl.ds(r0, ZC)],
                        num_out.at[c, pl.ds(r0, ZC)])
        return carry
    lax.fori_loop(0, STRIPE // ZC, _ex, 0)
    pltpu.sync_copy(den_v, den_out.at[c * 16 + s])


_sc_edge = pl.kernel(
    _sc_edge_body,
    out_type=[
        jax.ShapeDtypeStruct((HEADS, NPD, HID), jnp.float32),
        jax.ShapeDtypeStruct((32, NPD), jnp.float32),
    ],
    mesh=_sc_mesh,
    scratch_types=[
        pltpu.VMEM((2 * NPD,), jnp.float32),
        pltpu.VMEM((NPD,), jnp.float32),
        pltpu.VMEM((NCHUNK, CH), jnp.int32),
        pltpu.VMEM((NCHUNK, CH), jnp.int32),
        pltpu.VMEM((ZC, HID), jnp.float32),
        pltpu.VMEM((CH,), jnp.int32),
        pltpu.VMEM((1, CH + 16), jnp.float32),
        pltpu.VMEM((16,), jnp.float32),
        pltpu.VMEM_SHARED((NPD, HID), jnp.float32),
        pltpu.SemaphoreType.DMA,
    ],
    compiler_params=pltpu.CompilerParams(
        needs_layout_passes=False, use_tc_tiling_on_sc=False),
)


def _norm1_body(num_ref, den_ref, b1_ref, g_ref, bb_ref, rm_ref, rv_ref,
                w2_ref, a2_ref, x1_ref, h2_ref, asad2_ref, m2_ref):
    num = jnp.concatenate([num_ref[0, 0:N, :], num_ref[1, 0:N, :]], axis=1)
    dT = jnp.transpose(
        jnp.concatenate([jnp.sum(den_ref[0:16, :], axis=0, keepdims=True),
                         jnp.sum(den_ref[16:32, :], axis=0, keepdims=True)],
                        axis=0))
    d0 = jnp.broadcast_to(dT[0:N, 0:1], (N, HID))
    d1 = jnp.broadcast_to(dT[0:N, 1:2], (N, HID))
    den = jnp.concatenate([d0, d1], axis=1) + 1e-16
    x1 = num / den + b1_ref[...]
    x1 = g_ref[...] * (x1 - rm_ref[...]) / jnp.sqrt(rv_ref[...] + 1e-5) \
        + bb_ref[...]
    x1 = jnp.where(x1 > 0, x1, jnp.exp(x1) - 1.0)
    x1_ref[...] = x1
    h2 = jnp.dot(x1, w2_ref[...], preferred_element_type=jnp.float32)
    h2_ref[0, :, :] = h2[:, 0:HID]
    h2_ref[1, :, :] = h2[:, HID:2 * HID]
    asad = jnp.dot(h2, a2_ref[...], preferred_element_type=jnp.float32)
    asad2_ref[...] = asad
    mx = jnp.max(asad, axis=0, keepdims=True)
    ms = mx[:, 0:2] + mx[:, 2:4]
    ms = jnp.where(ms > 0, ms, 0.2 * ms)
    m2_ref[...] = jnp.concatenate(
        [ms, jnp.zeros((1, 14), jnp.float32)], axis=1)


def _norm2_body(x1_ref, num_ref, den_ref, b2_ref, wf_ref, bf_ref, o_ref):
    num = jnp.concatenate([num_ref[0, 0:N, :], num_ref[1, 0:N, :]], axis=1)
    dT = jnp.transpose(
        jnp.concatenate([jnp.sum(den_ref[0:16, :], axis=0, keepdims=True),
                         jnp.sum(den_ref[16:32, :], axis=0, keepdims=True)],
                        axis=0))
    d0 = jnp.broadcast_to(dT[0:N, 0:1], (N, HID))
    d1 = jnp.broadcast_to(dT[0:N, 1:2], (N, HID))
    den = jnp.concatenate([d0, d1], axis=1) + 1e-16
    x2 = num / den + b2_ref[...]
    xjk = jnp.maximum(x1_ref[...], x2)
    o_ref[...] = jnp.dot(xjk, wf_ref[...],
                         preferred_element_type=jnp.float32) + bf_ref[...]


def _pack_a(a_src, a_dst):
    z = jnp.zeros((HID,), jnp.float32)
    c0 = jnp.concatenate([a_src[0], z])
    c1 = jnp.concatenate([z, a_src[1]])
    c2 = jnp.concatenate([a_dst[0], z])
    c3 = jnp.concatenate([z, a_dst[1]])
    zc = jnp.zeros((HEADS * HID,), jnp.float32)
    return jnp.stack([c0, c1, c2, c3, zc, zc, zc, zc], axis=1)


def _prep_tables(h_split, asad, mv):
    hext = jnp.concatenate(
        [h_split, jnp.zeros((HEADS, NPD - N, HID), jnp.float32)], axis=1)
    hext = hext.reshape(HEADS * NPD, HID)
    t = jnp.concatenate(
        [asad[:, 0:4].T, jnp.full((4, NPD - N), -1e30, jnp.float32)], axis=1)
    tbl = jnp.stack([jnp.concatenate([t[0], t[2]]),
                     jnp.concatenate([t[1], t[3]])])
    return hext, tbl, mv.reshape(-1)


def kernel(x, edge_index, W1, a_src1, a_dst1, b1, bn_g, bn_b, bn_rm, bn_rv,
           W2, a_src2, a_dst2, b2, Wf, bf):
    loop = jnp.arange(N, dtype=edge_index.dtype)
    pad = jnp.full((EPAD - ET,), N, edge_index.dtype)
    src = jnp.concatenate([edge_index[0], loop, pad]).reshape(16, NCHUNK, CH)
    dst = jnp.concatenate([edge_index[1], loop, pad]).reshape(16, NCHUNK, CH)

    h1s, asad1, mv1 = _proj(x, W1, _pack_a(a_src1, a_dst1))
    h1e, tbl1, mv1 = _prep_tables(h1s, asad1, mv1)
    num1, den1 = _sc_edge(h1e, tbl1, mv1, src, dst)

    x1, h2s, asad2, mv2 = pl.pallas_call(
        _norm1_body,
        out_shape=[
            jax.ShapeDtypeStruct((N, HEADS * HID), jnp.float32),
            jax.ShapeDtypeStruct((HEADS, N, HID), jnp.float32),
            jax.ShapeDtypeStruct((N, 8), jnp.float32),
            jax.ShapeDtypeStruct((1, 16), jnp.float32),
        ],
    )(num1, den1, b1[None, :], bn_g[None, :], bn_b[None, :],
      bn_rm[None, :], bn_rv[None, :], W2, _pack_a(a_src2, a_dst2))

    h2e, tbl2, mv2 = _prep_tables(h2s, asad2, mv2)
    num2, den2 = _sc_edge(h2e, tbl2, mv2, src, dst)

    return pl.pallas_call(
        _norm2_body,
        out_shape=jax.ShapeDtypeStruct((N, OUT_CH), jnp.float32),
    )(x1, num2, den2, b2[None, :], Wf, bf[None, :])


# speedup vs baseline: 46.5036x; 1.0395x over previous
import jax
import jax.numpy as jnp
from jax import lax
from jax.experimental import pallas as pl
from jax.experimental.pallas import tpu as pltpu
from jax.experimental.pallas import tpu_sc as plsc

N = 10000
HID = 64
HEADS = 2
OUT_CH = 128

NPD = 10240
STRIPE = NPD // 16
ZC = 64
CH = 64
ET = N + 320000
EPT = 20736
NCHUNK = EPT // CH
EPAD = 16 * EPT


def _k1_body(x_ref, w_ref, a_ref, h_ref, asad_ref, m_ref):
    h = jnp.dot(x_ref[...], w_ref[...], preferred_element_type=jnp.float32)
    h_ref[0, :, :] = h[:, 0:HID]
    h_ref[1, :, :] = h[:, HID:2 * HID]
    asad = jnp.dot(h, a_ref[...], preferred_element_type=jnp.float32)
    asad_ref[...] = asad
    mx = jnp.max(asad, axis=0, keepdims=True)
    ms = mx[:, 0:2] + mx[:, 2:4]
    ms = jnp.where(ms > 0, ms, 0.2 * ms)
    m_ref[...] = jnp.concatenate(
        [ms, jnp.zeros((1, 14), jnp.float32)], axis=1)


def _proj(x, w, a):
    n = x.shape[0]
    return pl.pallas_call(
        _k1_body,
        out_shape=[
            jax.ShapeDtypeStruct((HEADS, n, HID), jnp.float32),
            jax.ShapeDtypeStruct((n, 8), jnp.float32),
            jax.ShapeDtypeStruct((1, 16), jnp.float32),
        ],
    )(x, w, a)


_sc_mesh = plsc.VectorSubcoreMesh(core_axis_name="c", subcore_axis_name="s")


def _sc_edge_body(h_hbm, tbl_hbm, mv_hbm, src_hbm, dst_hbm,
                  num_out, den_out,
                  tbl_v, den_v, sslab, dslab, rows_v, ibuf, pbuf, mv,
                  num_sh, sem):
    c = lax.axis_index("c")
    s = lax.axis_index("s")

    def _zb(j, carry):
        for cc in range(HID // 16):
            rows_v[j, pl.ds(cc * 16, 16)] = jnp.zeros((16,), jnp.float32)
        return carry
    lax.fori_loop(0, ZC, _zb, 0)

    def _zn(k, carry):
        pltpu.sync_copy(rows_v, num_sh.at[pl.ds(s * STRIPE + k * ZC, ZC)])
        return carry
    lax.fori_loop(0, STRIPE // ZC, _zn, 0)

    def _zd(i, carry):
        den_v[pl.ds(i * 16, 16)] = jnp.zeros((16,), jnp.float32)
        return carry
    lax.fori_loop(0, NPD // 16, _zd, 0)

    pltpu.sync_copy(tbl_hbm.at[c], tbl_v)
    pltpu.sync_copy(mv_hbm, mv)
    pltpu.sync_copy(src_hbm.at[s], sslab)
    pltpu.sync_copy(dst_hbm.at[s], dslab)
    mvv = mv[pl.ds(0, 16)]
    m = jnp.where(c == 0, mvv[0], mvv[1])
    hoff = c * NPD
    plsc.subcore_barrier()

    def _chunk(ci, carry):
        for g in range(CH // 16):
            s16 = sslab[ci, pl.ds(g * 16, 16)]
            ibuf[pl.ds(g * 16, 16)] = s16 + hoff
        cp = pltpu.async_copy(h_hbm.at[ibuf], rows_v, sem)
        for g in range(CH // 16):
            s16 = sslab[ci, pl.ds(g * 16, 16)]
            d16 = dslab[ci, pl.ds(g * 16, 16)]
            a_s = plsc.load_gather(tbl_v, [s16])
            a_d = plsc.load_gather(tbl_v, [d16 + NPD])
            e = a_s + a_d
            e = jnp.where(e > 0, e, 0.2 * e) - m
            p = jnp.exp(e)
            plsc.addupdate_scatter(den_v, [d16], p)
            pbuf[0, pl.ds(g * 16, 16)] = p
        cp.wait()

        def _scale(j, carry2):
            ps = pbuf[0, pl.ds(j, 16)][0]
            for cc in range(HID // 16):
                rows_v[j, pl.ds(cc * 16, 16)] = (
                    rows_v[j, pl.ds(cc * 16, 16)] * ps)
            return carry2
        lax.fori_loop(0, CH, _scale, 0)
        pltpu.sync_copy(rows_v, num_sh.at[dslab.at[ci]], add=True)
        return carry
    lax.fori_loop(0, NCHUNK, _chunk, 0)
    plsc.subcore_barrier()

    def _ex(k, carry):
        r0 = s * STRIPE + k * ZC
        pltpu.sync_copy(num_sh.at[pl.ds(r0, ZC)],
                        num_out.at[c, pl.ds(r0, ZC)])
        return carry
    lax.fori_loop(0, STRIPE // ZC, _ex, 0)
    pltpu.sync_copy(den_v, den_out.at[c * 16 + s])


_sc_edge = pl.kernel(
    _sc_edge_body,
    out_type=[
        jax.ShapeDtypeStruct((HEADS, NPD, HID), jnp.float32),
        jax.ShapeDtypeStruct((32, NPD), jnp.float32),
    ],
    mesh=_sc_mesh,
    scratch_types=[
        pltpu.VMEM((2 * NPD,), jnp.float32),
        pltpu.VMEM((NPD,), jnp.float32),
        pltpu.VMEM((NCHUNK, CH), jnp.int32),
        pltpu.VMEM((NCHUNK, CH), jnp.int32),
        pltpu.VMEM((ZC, HID), jnp.float32),
        pltpu.VMEM((CH,), jnp.int32),
        pltpu.VMEM((1, CH + 16), jnp.float32),
        pltpu.VMEM((16,), jnp.float32),
        pltpu.VMEM_SHARED((NPD, HID), jnp.float32),
        pltpu.SemaphoreType.DMA,
    ],
    compiler_params=pltpu.CompilerParams(
        needs_layout_passes=False, use_tc_tiling_on_sc=False),
)


def _norm1_body(num_ref, den_ref, b1_ref, g_ref, bb_ref, rm_ref, rv_ref,
                w2_ref, a2_ref, x1_ref, h2_ref, asad2_ref, m2_ref):
    num = jnp.concatenate([num_ref[0, 0:N, :], num_ref[1, 0:N, :]], axis=1)
    dT = jnp.transpose(
        jnp.concatenate([jnp.sum(den_ref[0:16, :], axis=0, keepdims=True),
                         jnp.sum(den_ref[16:32, :], axis=0, keepdims=True)],
                        axis=0))
    d0 = jnp.broadcast_to(dT[0:N, 0:1], (N, HID))
    d1 = jnp.broadcast_to(dT[0:N, 1:2], (N, HID))
    den = jnp.concatenate([d0, d1], axis=1) + 1e-16
    x1 = num / den + b1_ref[...]
    x1 = g_ref[...] * (x1 - rm_ref[...]) / jnp.sqrt(rv_ref[...] + 1e-5) \
        + bb_ref[...]
    x1 = jnp.where(x1 > 0, x1, jnp.exp(x1) - 1.0)
    x1_ref[...] = x1
    h2 = jnp.dot(x1, w2_ref[...], preferred_element_type=jnp.float32)
    h2_ref[0, :, :] = h2[:, 0:HID]
    h2_ref[1, :, :] = h2[:, HID:2 * HID]
    asad = jnp.dot(h2, a2_ref[...], preferred_element_type=jnp.float32)
    asad2_ref[...] = asad
    mx = jnp.max(asad, axis=0, keepdims=True)
    ms = mx[:, 0:2] + mx[:, 2:4]
    ms = jnp.where(ms > 0, ms, 0.2 * ms)
    m2_ref[...] = jnp.concatenate(
        [ms, jnp.zeros((1, 14), jnp.float32)], axis=1)


def _norm2_body(x1_ref, num_ref, den_ref, b2_ref, wf_ref, bf_ref, o_ref):
    num = jnp.concatenate([num_ref[0, 0:N, :], num_ref[1, 0:N, :]], axis=1)
    dT = jnp.transpose(
        jnp.concatenate([jnp.sum(den_ref[0:16, :], axis=0, keepdims=True),
                         jnp.sum(den_ref[16:32, :], axis=0, keepdims=True)],
                        axis=0))
    d0 = jnp.broadcast_to(dT[0:N, 0:1], (N, HID))
    d1 = jnp.broadcast_to(dT[0:N, 1:2], (N, HID))
    den = jnp.concatenate([d0, d1], axis=1) + 1e-16
    x2 = num / den + b2_ref[...]
    xjk = jnp.maximum(x1_ref[...], x2)
    o_ref[...] = jnp.dot(xjk, wf_ref[...],
                         preferred_element_type=jnp.float32) + bf_ref[...]


def _pack_a(a_src, a_dst):
    z = jnp.zeros((HID,), jnp.float32)
    c0 = jnp.concatenate([a_src[0], z])
    c1 = jnp.concatenate([z, a_src[1]])
    c2 = jnp.concatenate([a_dst[0], z])
    c3 = jnp.concatenate([z, a_dst[1]])
    zc = jnp.zeros((HEADS * HID,), jnp.float32)
    return jnp.stack([c0, c1, c2, c3, zc, zc, zc, zc], axis=1)


def _prep_tables(h_split, asad, mv):
    hext = jnp.concatenate(
        [h_split, jnp.zeros((HEADS, NPD - N, HID), jnp.float32)], axis=1)
    hext = hext.reshape(HEADS * NPD, HID)
    t = jnp.concatenate(
        [asad[:, 0:4].T, jnp.full((4, NPD - N), -1e30, jnp.float32)], axis=1)
    tbl = jnp.stack([jnp.concatenate([t[0], t[2]]),
                     jnp.concatenate([t[1], t[3]])])
    return hext, tbl, mv.reshape(-1)


def kernel(x, edge_index, W1, a_src1, a_dst1, b1, bn_g, bn_b, bn_rm, bn_rv,
           W2, a_src2, a_dst2, b2, Wf, bf):
    loop = jnp.arange(N, dtype=edge_index.dtype)
    pad = jnp.full((EPAD - ET,), N, edge_index.dtype)
    src = jnp.concatenate([edge_index[0], loop, pad]).reshape(16, NCHUNK, CH)
    dst = jnp.concatenate([edge_index[1], loop, pad]).reshape(16, NCHUNK, CH)

    h1s, asad1, mv1 = _proj(x, W1, _pack_a(a_src1, a_dst1))
    h1e, tbl1, mv1 = _prep_tables(h1s, asad1, mv1)
    num1, den1 = _sc_edge(h1e, tbl1, mv1, src, dst)

    x1, h2s, asad2, mv2 = pl.pallas_call(
        _norm1_body,
        out_shape=[
            jax.ShapeDtypeStruct((N, HEADS * HID), jnp.float32),
            jax.ShapeDtypeStruct((HEADS, N, HID), jnp.float32),
            jax.ShapeDtypeStruct((N, 8), jnp.float32),
            jax.ShapeDtypeStruct((1, 16), jnp.float32),
        ],
    )(num1, den1, b1[None, :], bn_g[None, :], bn_b[None, :],
      bn_rm[None, :], bn_rv[None, :], W2, _pack_a(a_src2, a_dst2))

    h2e, tbl2, mv2 = _prep_tables(h2s, asad2, mv2)
    num2, den2 = _sc_edge(h2e, tbl2, mv2, src, dst)

    return pl.pallas_call(
        _norm2_body,
        out_shape=jax.ShapeDtypeStruct((N, OUT_CH), jnp.float32),
    )(x1, num2, den2, b2[None, :], Wf, bf[None, :])


# speedup vs baseline: 78.4493x; 1.6870x over previous
import jax
import jax.numpy as jnp
from jax import lax
from jax.experimental import pallas as pl
from jax.experimental.pallas import tpu as pltpu
from jax.experimental.pallas import tpu_sc as plsc

N = 10000
HID = 64
HEADS = 2
OUT_CH = 128

NPD = 10240
STRIPE = NPD // 16
ZC = 64
CH = 64
ET = N + 320000
EPT = 20736
NCHUNK = EPT // CH
EPAD = 16 * EPT


def _k1_body(x_ref, w_ref, a_ref, h_ref, asad_ref, m_ref):
    h = jnp.dot(x_ref[...], w_ref[...], preferred_element_type=jnp.float32)
    h_ref[0, :, :] = h[:, 0:HID]
    h_ref[1, :, :] = h[:, HID:2 * HID]
    asad = jnp.dot(h, a_ref[...], preferred_element_type=jnp.float32)
    asad_ref[...] = asad
    mx = jnp.max(asad, axis=0, keepdims=True)
    ms = mx[:, 0:2] + mx[:, 2:4]
    ms = jnp.where(ms > 0, ms, 0.2 * ms)
    m_ref[...] = jnp.concatenate(
        [ms, jnp.zeros((1, 14), jnp.float32)], axis=1)


def _proj(x, w, a):
    n = x.shape[0]
    return pl.pallas_call(
        _k1_body,
        out_shape=[
            jax.ShapeDtypeStruct((HEADS, n, HID), jnp.float32),
            jax.ShapeDtypeStruct((n, 8), jnp.float32),
            jax.ShapeDtypeStruct((1, 16), jnp.float32),
        ],
    )(x, w, a)


_sc_mesh = plsc.VectorSubcoreMesh(core_axis_name="c", subcore_axis_name="s")


def _sc_edge_body(h_hbm, tbl_hbm, mv_hbm, src_hbm, dst_hbm,
                  num_out, den_out,
                  tbl_v, den_v, sslab, dslab, rows_v, rows_v2, ibuf, ibuf2,
                  pbuf, pbuf2, mv, num_sh, sem, sem2, sem3, sem4):
    c = lax.axis_index("c")
    s = lax.axis_index("s")

    def _zb(j, carry):
        for cc in range(HID // 16):
            rows_v[j, pl.ds(cc * 16, 16)] = jnp.zeros((16,), jnp.float32)
        return carry
    lax.fori_loop(0, ZC, _zb, 0)

    def _zn(k, carry):
        pltpu.sync_copy(rows_v, num_sh.at[pl.ds(s * STRIPE + k * ZC, ZC)])
        return carry
    lax.fori_loop(0, STRIPE // ZC, _zn, 0)

    def _zd(i, carry):
        den_v[pl.ds(i * 16, 16)] = jnp.zeros((16,), jnp.float32)
        return carry
    lax.fori_loop(0, NPD // 16, _zd, 0)

    pltpu.sync_copy(tbl_hbm.at[c], tbl_v)
    pltpu.sync_copy(mv_hbm, mv)
    pltpu.sync_copy(src_hbm.at[s], sslab)
    pltpu.sync_copy(dst_hbm.at[s], dslab)
    mvv = mv[pl.ds(0, 16)]
    m = jnp.where(c == 0, mvv[0], mvv[1])
    hoff = c * NPD
    plsc.subcore_barrier()

    def _ibuild(ci, ib):
        for g in range(CH // 16):
            ib[pl.ds(g * 16, 16)] = sslab[ci, pl.ds(g * 16, 16)] + hoff

    def _alphas(ci, pb):
        for g in range(CH // 16):
            s16 = sslab[ci, pl.ds(g * 16, 16)]
            d16 = dslab[ci, pl.ds(g * 16, 16)]
            a_s = plsc.load_gather(tbl_v, [s16])
            a_d = plsc.load_gather(tbl_v, [d16 + NPD])
            e = a_s + a_d
            e = jnp.where(e > 0, e, 0.2 * e) - m
            p = jnp.exp(e)
            plsc.addupdate_scatter(den_v, [d16], p)
            pb[0, pl.ds(g * 16, 16)] = p

    def _scale(rv, pb):
        for g in range(CH // 16):
            pv = pb[0, pl.ds(g * 16, 16)]
            for l in range(16):
                ps = pv[l]
                j = g * 16 + l
                for cc in range(HID // 16):
                    rv[j, pl.ds(cc * 16, 16)] = (
                        rv[j, pl.ds(cc * 16, 16)] * ps)

    def _pair(k, carry):
        ca = 2 * k
        cb = 2 * k + 1
        _ibuild(ca, ibuf)
        gA = pltpu.async_copy(h_hbm.at[ibuf], rows_v, sem)
        _alphas(ca, pbuf)
        _ibuild(cb, ibuf2)
        gB = pltpu.async_copy(h_hbm.at[ibuf2], rows_v2, sem2)
        gA.wait()
        _scale(rows_v, pbuf)
        sA = pltpu.async_copy(rows_v, num_sh.at[dslab.at[ca]], sem3,
                              add=True)
        _alphas(cb, pbuf2)
        gB.wait()
        _scale(rows_v2, pbuf2)
        sB = pltpu.async_copy(rows_v2, num_sh.at[dslab.at[cb]], sem4,
                              add=True)
        sA.wait()
        sB.wait()
        return carry
    lax.fori_loop(0, NCHUNK // 2, _pair, 0)
    plsc.subcore_barrier()

    def _ex(k, carry):
        r0 = s * STRIPE + k * ZC
        pltpu.sync_copy(num_sh.at[pl.ds(r0, ZC)],
                        num_out.at[c, pl.ds(r0, ZC)])
        return carry
    lax.fori_loop(0, STRIPE // ZC, _ex, 0)
    pltpu.sync_copy(den_v, den_out.at[c * 16 + s])


_sc_edge = pl.kernel(
    _sc_edge_body,
    out_type=[
        jax.ShapeDtypeStruct((HEADS, NPD, HID), jnp.float32),
        jax.ShapeDtypeStruct((32, NPD), jnp.float32),
    ],
    mesh=_sc_mesh,
    scratch_types=[
        pltpu.VMEM((2 * NPD,), jnp.float32),
        pltpu.VMEM((NPD,), jnp.float32),
        pltpu.VMEM((NCHUNK, CH), jnp.int32),
        pltpu.VMEM((NCHUNK, CH), jnp.int32),
        pltpu.VMEM((ZC, HID), jnp.float32),
        pltpu.VMEM((ZC, HID), jnp.float32),
        pltpu.VMEM((CH,), jnp.int32),
        pltpu.VMEM((CH,), jnp.int32),
        pltpu.VMEM((1, CH + 16), jnp.float32),
        pltpu.VMEM((1, CH + 16), jnp.float32),
        pltpu.VMEM((16,), jnp.float32),
        pltpu.VMEM_SHARED((NPD, HID), jnp.float32),
        pltpu.SemaphoreType.DMA,
        pltpu.SemaphoreType.DMA,
        pltpu.SemaphoreType.DMA,
        pltpu.SemaphoreType.DMA,
    ],
    compiler_params=pltpu.CompilerParams(
        needs_layout_passes=False, use_tc_tiling_on_sc=False),
)


def _norm1_body(num_ref, den_ref, b1_ref, g_ref, bb_ref, rm_ref, rv_ref,
                w2_ref, a2_ref, x1_ref, h2_ref, asad2_ref, m2_ref):
    num = jnp.concatenate([num_ref[0, 0:N, :], num_ref[1, 0:N, :]], axis=1)
    dT = jnp.transpose(
        jnp.concatenate([jnp.sum(den_ref[0:16, :], axis=0, keepdims=True),
                         jnp.sum(den_ref[16:32, :], axis=0, keepdims=True)],
                        axis=0))
    d0 = jnp.broadcast_to(dT[0:N, 0:1], (N, HID))
    d1 = jnp.broadcast_to(dT[0:N, 1:2], (N, HID))
    den = jnp.concatenate([d0, d1], axis=1) + 1e-16
    x1 = num / den + b1_ref[...]
    x1 = g_ref[...] * (x1 - rm_ref[...]) / jnp.sqrt(rv_ref[...] + 1e-5) \
        + bb_ref[...]
    x1 = jnp.where(x1 > 0, x1, jnp.exp(x1) - 1.0)
    x1_ref[...] = x1
    h2 = jnp.dot(x1, w2_ref[...], preferred_element_type=jnp.float32)
    h2_ref[0, :, :] = h2[:, 0:HID]
    h2_ref[1, :, :] = h2[:, HID:2 * HID]
    asad = jnp.dot(h2, a2_ref[...], preferred_element_type=jnp.float32)
    asad2_ref[...] = asad
    mx = jnp.max(asad, axis=0, keepdims=True)
    ms = mx[:, 0:2] + mx[:, 2:4]
    ms = jnp.where(ms > 0, ms, 0.2 * ms)
    m2_ref[...] = jnp.concatenate(
        [ms, jnp.zeros((1, 14), jnp.float32)], axis=1)


def _norm2_body(x1_ref, num_ref, den_ref, b2_ref, wf_ref, bf_ref, o_ref):
    num = jnp.concatenate([num_ref[0, 0:N, :], num_ref[1, 0:N, :]], axis=1)
    dT = jnp.transpose(
        jnp.concatenate([jnp.sum(den_ref[0:16, :], axis=0, keepdims=True),
                         jnp.sum(den_ref[16:32, :], axis=0, keepdims=True)],
                        axis=0))
    d0 = jnp.broadcast_to(dT[0:N, 0:1], (N, HID))
    d1 = jnp.broadcast_to(dT[0:N, 1:2], (N, HID))
    den = jnp.concatenate([d0, d1], axis=1) + 1e-16
    x2 = num / den + b2_ref[...]
    xjk = jnp.maximum(x1_ref[...], x2)
    o_ref[...] = jnp.dot(xjk, wf_ref[...],
                         preferred_element_type=jnp.float32) + bf_ref[...]


def _pack_a(a_src, a_dst):
    z = jnp.zeros((HID,), jnp.float32)
    c0 = jnp.concatenate([a_src[0], z])
    c1 = jnp.concatenate([z, a_src[1]])
    c2 = jnp.concatenate([a_dst[0], z])
    c3 = jnp.concatenate([z, a_dst[1]])
    zc = jnp.zeros((HEADS * HID,), jnp.float32)
    return jnp.stack([c0, c1, c2, c3, zc, zc, zc, zc], axis=1)


def _prep_tables(h_split, asad, mv):
    hext = jnp.concatenate(
        [h_split, jnp.zeros((HEADS, NPD - N, HID), jnp.float32)], axis=1)
    hext = hext.reshape(HEADS * NPD, HID)
    t = jnp.concatenate(
        [asad[:, 0:4].T, jnp.full((4, NPD - N), -1e30, jnp.float32)], axis=1)
    tbl = jnp.stack([jnp.concatenate([t[0], t[2]]),
                     jnp.concatenate([t[1], t[3]])])
    return hext, tbl, mv.reshape(-1)


def kernel(x, edge_index, W1, a_src1, a_dst1, b1, bn_g, bn_b, bn_rm, bn_rv,
           W2, a_src2, a_dst2, b2, Wf, bf):
    loop = jnp.arange(N, dtype=edge_index.dtype)
    pad = jnp.full((EPAD - ET,), N, edge_index.dtype)
    src = jnp.concatenate([edge_index[0], loop, pad]).reshape(16, NCHUNK, CH)
    dst = jnp.concatenate([edge_index[1], loop, pad]).reshape(16, NCHUNK, CH)

    h1s, asad1, mv1 = _proj(x, W1, _pack_a(a_src1, a_dst1))
    h1e, tbl1, mv1 = _prep_tables(h1s, asad1, mv1)
    num1, den1 = _sc_edge(h1e, tbl1, mv1, src, dst)

    x1, h2s, asad2, mv2 = pl.pallas_call(
        _norm1_body,
        out_shape=[
            jax.ShapeDtypeStruct((N, HEADS * HID), jnp.float32),
            jax.ShapeDtypeStruct((HEADS, N, HID), jnp.float32),
            jax.ShapeDtypeStruct((N, 8), jnp.float32),
            jax.ShapeDtypeStruct((1, 16), jnp.float32),
        ],
    )(num1, den1, b1[None, :], bn_g[None, :], bn_b[None, :],
      bn_rm[None, :], bn_rv[None, :], W2, _pack_a(a_src2, a_dst2))

    h2e, tbl2, mv2 = _prep_tables(h2s, asad2, mv2)
    num2, den2 = _sc_edge(h2e, tbl2, mv2, src, dst)

    return pl.pallas_call(
        _norm2_body,
        out_shape=jax.ShapeDtypeStruct((N, OUT_CH), jnp.float32),
    )(x1, num2, den2, b2[None, :], Wf, bf[None, :])


# speedup vs baseline: 80.6140x; 1.0276x over previous
import jax
import jax.numpy as jnp
from jax import lax
from jax.experimental import pallas as pl
from jax.experimental.pallas import tpu as pltpu
from jax.experimental.pallas import tpu_sc as plsc

N = 10000
HID = 64
HEADS = 2
OUT_CH = 128

NPD = 10240
STRIPE = NPD // 16
ZC = 64
CH = 64
ET = N + 320000
EPT = 20736
NCHUNK = EPT // CH
EPAD = 16 * EPT


def _emit_tables(h, asad, h_ref, tbl_ref, m_ref):
    h_ref[0, 0:N, :] = h[:, 0:HID]
    h_ref[1, 0:N, :] = h[:, HID:2 * HID]
    zpad = jnp.zeros((NPD - N, HID), jnp.float32)
    h_ref[0, N:NPD, :] = zpad
    h_ref[1, N:NPD, :] = zpad
    t = jnp.transpose(asad[:, 0:4])
    t = jnp.concatenate(
        [t, jnp.full((4, NPD - N), -1e30, jnp.float32)], axis=1)
    tbl_ref[...] = jnp.concatenate(
        [jnp.concatenate([t[0:1], t[2:3]], axis=1),
         jnp.concatenate([t[1:2], t[3:4]], axis=1)], axis=0)
    mx = jnp.max(asad, axis=0, keepdims=True)
    ms = mx[:, 0:2] + mx[:, 2:4]
    ms = jnp.where(ms > 0, ms, 0.2 * ms)
    m_ref[...] = jnp.concatenate(
        [ms, jnp.zeros((1, 14), jnp.float32)], axis=1)


def _k1_body(x_ref, w_ref, a_ref, h_ref, tbl_ref, m_ref):
    h = jnp.dot(x_ref[...], w_ref[...], preferred_element_type=jnp.float32)
    asad = jnp.dot(h, a_ref[...], preferred_element_type=jnp.float32)
    _emit_tables(h, asad, h_ref, tbl_ref, m_ref)


def _proj(x, w, a):
    return pl.pallas_call(
        _k1_body,
        out_shape=[
            jax.ShapeDtypeStruct((HEADS, NPD, HID), jnp.float32),
            jax.ShapeDtypeStruct((2, 2 * NPD), jnp.float32),
            jax.ShapeDtypeStruct((1, 16), jnp.float32),
        ],
    )(x, w, a)


_sc_mesh = plsc.VectorSubcoreMesh(core_axis_name="c", subcore_axis_name="s")


def _sc_edge_body(h_hbm, tbl_hbm, mv_hbm, src_hbm, dst_hbm,
                  num_out, den_out,
                  tbl_v, den_v, sslab, dslab, rows_v, rows_v2, ibuf, ibuf2,
                  pbuf, pbuf2, mv, num_sh, sem, sem2, sem3, sem4):
    c = lax.axis_index("c")
    s = lax.axis_index("s")

    def _zb(j, carry):
        for cc in range(HID // 16):
            rows_v[j, pl.ds(cc * 16, 16)] = jnp.zeros((16,), jnp.float32)
        return carry
    lax.fori_loop(0, ZC, _zb, 0)

    def _zn(k, carry):
        pltpu.sync_copy(rows_v, num_sh.at[pl.ds(s * STRIPE + k * ZC, ZC)])
        return carry
    lax.fori_loop(0, STRIPE // ZC, _zn, 0)

    def _zd(i, carry):
        den_v[pl.ds(i * 16, 16)] = jnp.zeros((16,), jnp.float32)
        return carry
    lax.fori_loop(0, NPD // 16, _zd, 0)

    pltpu.sync_copy(tbl_hbm.at[c], tbl_v)
    pltpu.sync_copy(mv_hbm, mv)
    pltpu.sync_copy(src_hbm.at[s], sslab)
    pltpu.sync_copy(dst_hbm.at[s], dslab)
    mvv = mv[pl.ds(0, 16)]
    m = jnp.where(c == 0, mvv[0], mvv[1])
    hoff = c * NPD
    plsc.subcore_barrier()

    def _ibuild(ci, ib):
        for g in range(CH // 16):
            ib[pl.ds(g * 16, 16)] = sslab[ci, pl.ds(g * 16, 16)] + hoff

    def _alphas(ci, pb):
        for g in range(CH // 16):
            s16 = sslab[ci, pl.ds(g * 16, 16)]
            d16 = dslab[ci, pl.ds(g * 16, 16)]
            a_s = plsc.load_gather(tbl_v, [s16])
            a_d = plsc.load_gather(tbl_v, [d16 + NPD])
            e = a_s + a_d
            e = jnp.where(e > 0, e, 0.2 * e) - m
            p = jnp.exp(e)
            plsc.addupdate_scatter(den_v, [d16], p)
            pb[0, pl.ds(g * 16, 16)] = p

    def _scale(rv, pb):
        for g in range(CH // 16):
            pv = pb[0, pl.ds(g * 16, 16)]
            for l in range(16):
                ps = pv[l]
                j = g * 16 + l
                for cc in range(HID // 16):
                    rv[j, pl.ds(cc * 16, 16)] = (
                        rv[j, pl.ds(cc * 16, 16)] * ps)

    def _pair(k, carry):
        ca = 2 * k
        cb = 2 * k + 1
        _ibuild(ca, ibuf)
        gA = pltpu.async_copy(h_hbm.at[ibuf], rows_v, sem)
        _ibuild(cb, ibuf2)
        gB = pltpu.async_copy(h_hbm.at[ibuf2], rows_v2, sem2)
        _alphas(ca, pbuf)
        gA.wait()
        _scale(rows_v, pbuf)
        sA = pltpu.async_copy(rows_v, num_sh.at[dslab.at[ca]], sem3,
                              add=True)
        _alphas(cb, pbuf2)
        gB.wait()
        _scale(rows_v2, pbuf2)
        sB = pltpu.async_copy(rows_v2, num_sh.at[dslab.at[cb]], sem4,
                              add=True)
        sA.wait()
        sB.wait()
        return carry
    lax.fori_loop(0, NCHUNK // 2, _pair, 0)
    plsc.subcore_barrier()

    def _ex(k, carry):
        r0 = s * STRIPE + k * ZC
        pltpu.sync_copy(num_sh.at[pl.ds(r0, ZC)],
                        num_out.at[c, pl.ds(r0, ZC)])
        return carry
    lax.fori_loop(0, STRIPE // ZC, _ex, 0)
    pltpu.sync_copy(den_v, den_out.at[c * 16 + s])


_sc_edge = pl.kernel(
    _sc_edge_body,
    out_type=[
        jax.ShapeDtypeStruct((HEADS, NPD, HID), jnp.float32),
        jax.ShapeDtypeStruct((32, NPD), jnp.float32),
    ],
    mesh=_sc_mesh,
    scratch_types=[
        pltpu.VMEM((2 * NPD,), jnp.float32),
        pltpu.VMEM((NPD,), jnp.float32),
        pltpu.VMEM((NCHUNK, CH), jnp.int32),
        pltpu.VMEM((NCHUNK, CH), jnp.int32),
        pltpu.VMEM((ZC, HID), jnp.float32),
        pltpu.VMEM((ZC, HID), jnp.float32),
        pltpu.VMEM((CH,), jnp.int32),
        pltpu.VMEM((CH,), jnp.int32),
        pltpu.VMEM((1, CH + 16), jnp.float32),
        pltpu.VMEM((1, CH + 16), jnp.float32),
        pltpu.VMEM((16,), jnp.float32),
        pltpu.VMEM_SHARED((NPD, HID), jnp.float32),
        pltpu.SemaphoreType.DMA,
        pltpu.SemaphoreType.DMA,
        pltpu.SemaphoreType.DMA,
        pltpu.SemaphoreType.DMA,
    ],
    compiler_params=pltpu.CompilerParams(
        needs_layout_passes=False, use_tc_tiling_on_sc=False),
)


def _norm1_body(num_ref, den_ref, b1_ref, g_ref, bb_ref, rm_ref, rv_ref,
                w2_ref, a2_ref, x1_ref, h2_ref, tbl2_ref, m2_ref):
    num = jnp.concatenate([num_ref[0, 0:N, :], num_ref[1, 0:N, :]], axis=1)
    dT = jnp.transpose(
        jnp.concatenate([jnp.sum(den_ref[0:16, :], axis=0, keepdims=True),
                         jnp.sum(den_ref[16:32, :], axis=0, keepdims=True)],
                        axis=0))
    d0 = jnp.broadcast_to(dT[0:N, 0:1], (N, HID))
    d1 = jnp.broadcast_to(dT[0:N, 1:2], (N, HID))
    den = jnp.concatenate([d0, d1], axis=1) + 1e-16
    x1 = num / den + b1_ref[...]
    x1 = g_ref[...] * (x1 - rm_ref[...]) / jnp.sqrt(rv_ref[...] + 1e-5) \
        + bb_ref[...]
    x1 = jnp.where(x1 > 0, x1, jnp.exp(x1) - 1.0)
    x1_ref[...] = x1
    h2 = jnp.dot(x1, w2_ref[...], preferred_element_type=jnp.float32)
    asad = jnp.dot(h2, a2_ref[...], preferred_element_type=jnp.float32)
    _emit_tables(h2, asad, h2_ref, tbl2_ref, m2_ref)


def _norm2_body(x1_ref, num_ref, den_ref, b2_ref, wf_ref, bf_ref, o_ref):
    num = jnp.concatenate([num_ref[0, 0:N, :], num_ref[1, 0:N, :]], axis=1)
    dT = jnp.transpose(
        jnp.concatenate([jnp.sum(den_ref[0:16, :], axis=0, keepdims=True),
                         jnp.sum(den_ref[16:32, :], axis=0, keepdims=True)],
                        axis=0))
    d0 = jnp.broadcast_to(dT[0:N, 0:1], (N, HID))
    d1 = jnp.broadcast_to(dT[0:N, 1:2], (N, HID))
    den = jnp.concatenate([d0, d1], axis=1) + 1e-16
    x2 = num / den + b2_ref[...]
    xjk = jnp.maximum(x1_ref[...], x2)
    o_ref[...] = jnp.dot(xjk, wf_ref[...],
                         preferred_element_type=jnp.float32) + bf_ref[...]


def _pack_a(a_src, a_dst):
    z = jnp.zeros((HID,), jnp.float32)
    c0 = jnp.concatenate([a_src[0], z])
    c1 = jnp.concatenate([z, a_src[1]])
    c2 = jnp.concatenate([a_dst[0], z])
    c3 = jnp.concatenate([z, a_dst[1]])
    zc = jnp.zeros((HEADS * HID,), jnp.float32)
    return jnp.stack([c0, c1, c2, c3, zc, zc, zc, zc], axis=1)


def kernel(x, edge_index, W1, a_src1, a_dst1, b1, bn_g, bn_b, bn_rm, bn_rv,
           W2, a_src2, a_dst2, b2, Wf, bf):
    loop = jnp.arange(N, dtype=edge_index.dtype)
    pad = jnp.full((EPAD - ET,), N, edge_index.dtype)
    src = jnp.concatenate([edge_index[0], loop, pad]).reshape(16, NCHUNK, CH)
    dst = jnp.concatenate([edge_index[1], loop, pad]).reshape(16, NCHUNK, CH)

    h1e, tbl1, mv1 = _proj(x, W1, _pack_a(a_src1, a_dst1))
    num1, den1 = _sc_edge(h1e.reshape(HEADS * NPD, HID), tbl1,
                          mv1.reshape(-1), src, dst)

    x1, h2e, tbl2, mv2 = pl.pallas_call(
        _norm1_body,
        out_shape=[
            jax.ShapeDtypeStruct((N, HEADS * HID), jnp.float32),
            jax.ShapeDtypeStruct((HEADS, NPD, HID), jnp.float32),
            jax.ShapeDtypeStruct((2, 2 * NPD), jnp.float32),
            jax.ShapeDtypeStruct((1, 16), jnp.float32),
        ],
    )(num1, den1, b1[None, :], bn_g[None, :], bn_b[None, :],
      bn_rm[None, :], bn_rv[None, :], W2, _pack_a(a_src2, a_dst2))

    num2, den2 = _sc_edge(h2e.reshape(HEADS * NPD, HID), tbl2,
                          mv2.reshape(-1), src, dst)

    return pl.pallas_call(
        _norm2_body,
        out_shape=jax.ShapeDtypeStruct((N, OUT_CH), jnp.float32),
    )(x1, num2, den2, b2[None, :], Wf, bf[None, :])


# speedup vs baseline: 90.5247x; 1.1229x over previous
import jax
import jax.numpy as jnp
from jax import lax
from jax.experimental import pallas as pl
from jax.experimental.pallas import tpu as pltpu
from jax.experimental.pallas import tpu_sc as plsc

N = 10000
HID = 64
HEADS = 2
OUT_CH = 128

NPD = 10240
STRIPE = NPD // 16
ZC = 64
CH = 64
ET = N + 320000
EPT = 20736
NCHUNK = EPT // CH
EPAD = 16 * EPT


def _emit_tables(h, asad, h_ref, tbl_ref, m_ref):
    h_ref[0, 0:N, :] = h[:, 0:HID]
    h_ref[1, 0:N, :] = h[:, HID:2 * HID]
    zpad = jnp.zeros((NPD - N, HID), jnp.float32)
    h_ref[0, N:NPD, :] = zpad
    h_ref[1, N:NPD, :] = zpad
    t = jnp.transpose(asad[:, 0:4])
    t = jnp.concatenate(
        [t, jnp.full((4, NPD - N), -1e30, jnp.float32)], axis=1)
    tbl_ref[...] = jnp.concatenate(
        [jnp.concatenate([t[0:1], t[2:3]], axis=1),
         jnp.concatenate([t[1:2], t[3:4]], axis=1)], axis=0)
    mx = jnp.max(asad, axis=0, keepdims=True)
    ms = mx[:, 0:2] + mx[:, 2:4]
    ms = jnp.where(ms > 0, ms, 0.2 * ms)
    m_ref[...] = jnp.concatenate(
        [ms, jnp.zeros((1, 14), jnp.float32)], axis=1)


def _k1_body(x_ref, w_ref, a_ref, h_ref, tbl_ref, m_ref):
    h = jnp.dot(x_ref[...], w_ref[...], preferred_element_type=jnp.float32)
    asad = jnp.dot(h, a_ref[...], preferred_element_type=jnp.float32)
    _emit_tables(h, asad, h_ref, tbl_ref, m_ref)


def _proj(x, w, a):
    return pl.pallas_call(
        _k1_body,
        out_shape=[
            jax.ShapeDtypeStruct((HEADS, NPD, HID), jnp.float32),
            jax.ShapeDtypeStruct((2, 2 * NPD), jnp.float32),
            jax.ShapeDtypeStruct((1, 16), jnp.float32),
        ],
    )(x, w, a)


_sc_mesh = plsc.VectorSubcoreMesh(core_axis_name="c", subcore_axis_name="s")


def _sc_edge_body(h_hbm, tbl_hbm, mv_hbm, src_hbm, dst_hbm,
                  num_out, den_out,
                  tbl_v, den_v, sslab, dslab, rows_v, rows_v2, ibuf, ibuf2,
                  pbuf, pbuf2, mv, num_sh, sem, sem2, sem3, sem4):
    c = lax.axis_index("c")
    s = lax.axis_index("s")

    def _zb(j, carry):
        for cc in range(HID // 16):
            rows_v[j, pl.ds(cc * 16, 16)] = jnp.zeros((16,), jnp.float32)
        return carry
    lax.fori_loop(0, ZC, _zb, 0)

    def _zn(k, carry):
        pltpu.sync_copy(rows_v, num_sh.at[pl.ds(s * STRIPE + k * ZC, ZC)])
        return carry
    lax.fori_loop(0, STRIPE // ZC, _zn, 0)

    def _zd(i, carry):
        den_v[pl.ds(i * 16, 16)] = jnp.zeros((16,), jnp.float32)
        return carry
    lax.fori_loop(0, NPD // 16, _zd, 0)

    pltpu.sync_copy(tbl_hbm.at[c], tbl_v)
    pltpu.sync_copy(mv_hbm, mv)
    pltpu.sync_copy(src_hbm.at[s], sslab)
    pltpu.sync_copy(dst_hbm.at[s], dslab)
    mvv = mv[pl.ds(0, 16)]
    m = jnp.where(c == 0, mvv[0], mvv[1])
    hoff = c * NPD
    plsc.subcore_barrier()

    def _ibuild(ci, ib):
        for g in range(CH // 16):
            ib[pl.ds(g * 16, 16)] = sslab[ci, pl.ds(g * 16, 16)] + hoff

    def _alphas(ci, pb):
        for g in range(CH // 16):
            s16 = sslab[ci, pl.ds(g * 16, 16)]
            d16 = dslab[ci, pl.ds(g * 16, 16)]
            a_s = plsc.load_gather(tbl_v, [s16])
            a_d = plsc.load_gather(tbl_v, [d16 + NPD])
            e = a_s + a_d
            e = jnp.where(e > 0, e, 0.2 * e) - m
            p = jnp.exp(e)
            plsc.addupdate_scatter(den_v, [d16], p)
            pb[0, pl.ds(g * 16, 16)] = p

    def _scale(rv, pb):
        for g in range(CH // 16):
            pv = pb[0, pl.ds(g * 16, 16)]
            for l in range(16):
                ps = pv[l]
                j = g * 16 + l
                for cc in range(HID // 16):
                    rv[j, pl.ds(cc * 16, 16)] = (
                        rv[j, pl.ds(cc * 16, 16)] * ps)

    def _pair(k, carry):
        ca = 2 * k
        cb = 2 * k + 1
        _ibuild(ca, ibuf)

        @pl.when(k > 0)
        def _drainA():
            pltpu.make_async_copy(
                rows_v, num_sh.at[dslab.at[ca]], sem3).wait()
        gA = pltpu.async_copy(h_hbm.at[ibuf], rows_v, sem)
        _ibuild(cb, ibuf2)

        @pl.when(k > 0)
        def _drainB():
            pltpu.make_async_copy(
                rows_v2, num_sh.at[dslab.at[cb]], sem4).wait()
        gB = pltpu.async_copy(h_hbm.at[ibuf2], rows_v2, sem2)
        _alphas(ca, pbuf)
        gA.wait()
        _scale(rows_v, pbuf)
        pltpu.async_copy(rows_v, num_sh.at[dslab.at[ca]], sem3, add=True)
        _alphas(cb, pbuf2)
        gB.wait()
        _scale(rows_v2, pbuf2)
        pltpu.async_copy(rows_v2, num_sh.at[dslab.at[cb]], sem4, add=True)
        return carry
    lax.fori_loop(0, NCHUNK // 2, _pair, 0)
    pltpu.make_async_copy(rows_v, num_sh.at[dslab.at[0]], sem3).wait()
    pltpu.make_async_copy(rows_v2, num_sh.at[dslab.at[0]], sem4).wait()
    plsc.subcore_barrier()

    def _ex(k, carry):
        r0 = s * STRIPE + k * ZC
        pltpu.sync_copy(num_sh.at[pl.ds(r0, ZC)],
                        num_out.at[c, pl.ds(r0, ZC)])
        return carry
    lax.fori_loop(0, STRIPE // ZC, _ex, 0)
    pltpu.sync_copy(den_v, den_out.at[c * 16 + s])


_sc_edge = pl.kernel(
    _sc_edge_body,
    out_type=[
        jax.ShapeDtypeStruct((HEADS, NPD, HID), jnp.float32),
        jax.ShapeDtypeStruct((32, NPD), jnp.float32),
    ],
    mesh=_sc_mesh,
    scratch_types=[
        pltpu.VMEM((2 * NPD,), jnp.float32),
        pltpu.VMEM((NPD,), jnp.float32),
        pltpu.VMEM((NCHUNK, CH), jnp.int32),
        pltpu.VMEM((NCHUNK, CH), jnp.int32),
        pltpu.VMEM((ZC, HID), jnp.float32),
        pltpu.VMEM((ZC, HID), jnp.float32),
        pltpu.VMEM((CH,), jnp.int32),
        pltpu.VMEM((CH,), jnp.int32),
        pltpu.VMEM((1, CH + 16), jnp.float32),
        pltpu.VMEM((1, CH + 16), jnp.float32),
        pltpu.VMEM((16,), jnp.float32),
        pltpu.VMEM_SHARED((NPD, HID), jnp.float32),
        pltpu.SemaphoreType.DMA,
        pltpu.SemaphoreType.DMA,
        pltpu.SemaphoreType.DMA,
        pltpu.SemaphoreType.DMA,
    ],
    compiler_params=pltpu.CompilerParams(
        needs_layout_passes=False, use_tc_tiling_on_sc=False),
)


def _norm1_body(num_ref, den_ref, b1_ref, g_ref, bb_ref, rm_ref, rv_ref,
                w2_ref, a2_ref, x1_ref, h2_ref, tbl2_ref, m2_ref):
    num = jnp.concatenate([num_ref[0, 0:N, :], num_ref[1, 0:N, :]], axis=1)
    dT = jnp.transpose(
        jnp.concatenate([jnp.sum(den_ref[0:16, :], axis=0, keepdims=True),
                         jnp.sum(den_ref[16:32, :], axis=0, keepdims=True)],
                        axis=0))
    d0 = jnp.broadcast_to(dT[0:N, 0:1], (N, HID))
    d1 = jnp.broadcast_to(dT[0:N, 1:2], (N, HID))
    den = jnp.concatenate([d0, d1], axis=1) + 1e-16
    x1 = num / den + b1_ref[...]
    x1 = g_ref[...] * (x1 - rm_ref[...]) / jnp.sqrt(rv_ref[...] + 1e-5) \
        + bb_ref[...]
    x1 = jnp.where(x1 > 0, x1, jnp.exp(x1) - 1.0)
    x1_ref[...] = x1
    h2 = jnp.dot(x1, w2_ref[...], preferred_element_type=jnp.float32)
    asad = jnp.dot(h2, a2_ref[...], preferred_element_type=jnp.float32)
    _emit_tables(h2, asad, h2_ref, tbl2_ref, m2_ref)


def _norm2_body(x1_ref, num_ref, den_ref, b2_ref, wf_ref, bf_ref, o_ref):
    num = jnp.concatenate([num_ref[0, 0:N, :], num_ref[1, 0:N, :]], axis=1)
    dT = jnp.transpose(
        jnp.concatenate([jnp.sum(den_ref[0:16, :], axis=0, keepdims=True),
                         jnp.sum(den_ref[16:32, :], axis=0, keepdims=True)],
                        axis=0))
    d0 = jnp.broadcast_to(dT[0:N, 0:1], (N, HID))
    d1 = jnp.broadcast_to(dT[0:N, 1:2], (N, HID))
    den = jnp.concatenate([d0, d1], axis=1) + 1e-16
    x2 = num / den + b2_ref[...]
    xjk = jnp.maximum(x1_ref[...], x2)
    o_ref[...] = jnp.dot(xjk, wf_ref[...],
                         preferred_element_type=jnp.float32) + bf_ref[...]


def _pack_a(a_src, a_dst):
    z = jnp.zeros((HID,), jnp.float32)
    c0 = jnp.concatenate([a_src[0], z])
    c1 = jnp.concatenate([z, a_src[1]])
    c2 = jnp.concatenate([a_dst[0], z])
    c3 = jnp.concatenate([z, a_dst[1]])
    zc = jnp.zeros((HEADS * HID,), jnp.float32)
    return jnp.stack([c0, c1, c2, c3, zc, zc, zc, zc], axis=1)


def kernel(x, edge_index, W1, a_src1, a_dst1, b1, bn_g, bn_b, bn_rm, bn_rv,
           W2, a_src2, a_dst2, b2, Wf, bf):
    loop = jnp.arange(N, dtype=edge_index.dtype)
    pad = jnp.full((EPAD - ET,), N, edge_index.dtype)
    src = jnp.concatenate([edge_index[0], loop, pad]).reshape(16, NCHUNK, CH)
    dst = jnp.concatenate([edge_index[1], loop, pad]).reshape(16, NCHUNK, CH)

    h1e, tbl1, mv1 = _proj(x, W1, _pack_a(a_src1, a_dst1))
    num1, den1 = _sc_edge(h1e.reshape(HEADS * NPD, HID), tbl1,
                          mv1.reshape(-1), src, dst)

    x1, h2e, tbl2, mv2 = pl.pallas_call(
        _norm1_body,
        out_shape=[
            jax.ShapeDtypeStruct((N, HEADS * HID), jnp.float32),
            jax.ShapeDtypeStruct((HEADS, NPD, HID), jnp.float32),
            jax.ShapeDtypeStruct((2, 2 * NPD), jnp.float32),
            jax.ShapeDtypeStruct((1, 16), jnp.float32),
        ],
    )(num1, den1, b1[None, :], bn_g[None, :], bn_b[None, :],
      bn_rm[None, :], bn_rv[None, :], W2, _pack_a(a_src2, a_dst2))

    num2, den2 = _sc_edge(h2e.reshape(HEADS * NPD, HID), tbl2,
                          mv2.reshape(-1), src, dst)

    return pl.pallas_call(
        _norm2_body,
        out_shape=jax.ShapeDtypeStruct((N, OUT_CH), jnp.float32),
    )(x1, num2, den2, b2[None, :], Wf, bf[None, :])
